# Initial kernel scaffold; baseline (speedup 1.0000x reference)
import jax
import jax.numpy as jnp
from jax.experimental import pallas as pl


def kernel(flow_traffic, flow_packets, flow_packet_size, flow_type, flow_p90PktSize, flow_bitrate_per_burst, flow_pkts_per_burst, flow_ipg_mean, flow_ipg_var, flow_on_rate, link_capacity, fe_W1, fe_b1, fe_W2, fe_b2, le_W1, le_b1, le_W2, le_b2, att_W, att_b, pu_Wi, pu_Wh, pu_bi, pu_bh, lu_Wi, lu_Wh, lu_bi, lu_bh, ro_W1, ro_b1, ro_W2, ro_b2, ro_W3, ro_b3, link_to_path, path_to_link_flow, path_to_link_pos):
    raise NotImplementedError("write your pallas kernel here")


# speedup vs baseline: 3.0970x; 3.0970x over previous
import functools

import jax
import jax.numpy as jnp
from jax import lax
from jax.experimental import pallas as pl
from jax.experimental.pallas import tpu as pltpu
from jax.experimental.pallas import tpu_sc as plsc

F = 16384
L = 4096
P = 8
D = 32
DIM = 16
ITERS = 12

NC, NS, LANES = 2, 16, 16
NW = NC * NS
M = F * P
MPW = M // NW
CH = 128
NCH = MPW // CH


_SELU_SCALE = 1.0507009873554805
_SELU_ALPHA = 1.6732632423543772


def _selu(x):
    return _SELU_SCALE * jnp.where(x > 0, x, _SELU_ALPHA * (jnp.exp(x) - 1.0))


def _softplus(x):
    return jnp.maximum(x, 0.0) + jnp.log(1.0 + jnp.exp(-jnp.abs(x)))


def _mesh():
    return plsc.VectorSubcoreMesh(core_axis_name="c", subcore_axis_name="s")


def _wid():
    return lax.axis_index("s") * NC + lax.axis_index("c")


def _sc_gather_body(table_hbm, idx_hbm, out_hbm, idx_v, rows_v, sem):
    w = _wid()
    pltpu.sync_copy(idx_hbm.at[pl.ds(w * NCH, NCH)], idx_v)
    cps = [pltpu.async_copy(table_hbm.at[idx_v.at[j]],
                            rows_v.at[pl.ds(j * CH, CH)], sem)
           for j in range(NCH)]
    for c in cps:
        c.wait()
    pltpu.sync_copy(rows_v, out_hbm.at[pl.ds(w * MPW, MPW)])


def _sc_gather(table, idx2d):
    k = pl.kernel(
        _sc_gather_body,
        out_type=jax.ShapeDtypeStruct((M, DIM), jnp.float32),
        mesh=_mesh(),
        scratch_types=[
            pltpu.VMEM((NCH, CH), jnp.int32),
            pltpu.VMEM((MPW, DIM), jnp.float32),
            pltpu.SemaphoreType.DMA,
        ],
        compiler_params=pltpu.CompilerParams(use_tc_tiling_on_sc=False),
    )
    return k(table, idx2d)


_LPW = L // NW


def _sc_prolog_body(tr_hbm, cap_hbm, pf_hbm, pp_hbm, ltp_hbm,
                    loadsum_hbm, capg_hbm, flat2_hbm,
                    tr_v, cap_v, pf_v, pp_v, ltp_v, ls_v, cg_v, f2_v):
    w = _wid()
    base = w * MPW
    pltpu.sync_copy(tr_hbm, tr_v)
    pltpu.sync_copy(cap_hbm, cap_v)
    pltpu.sync_copy(pf_hbm.at[pl.ds(base, MPW)], pf_v)
    pltpu.sync_copy(pp_hbm.at[pl.ds(base, MPW)], pp_v)
    pltpu.sync_copy(ltp_hbm.at[pl.ds(base, MPW)], ltp_v)

    lanes = lax.iota(jnp.int32, LANES)
    for g in range(_LPW // LANES):
        acc = jnp.zeros((LANES,), jnp.float32)
        for d in range(D):
            pos = lanes * D + (g * LANES * D + d)
            fidx = plsc.load_gather(pf_v, [pos])
            acc = acc + plsc.load_gather(tr_v, [fidx])
        ls_v[pl.ds(g * LANES, LANES)] = acc
    for k in range(MPW // LANES):
        sl = pl.ds(k * LANES, LANES)
        li = ltp_v[sl]
        cg_v[sl] = plsc.load_gather(cap_v, [li])
        f2_v[sl] = pp_v[sl] * F + pf_v[sl]

    pltpu.sync_copy(ls_v, loadsum_hbm.at[pl.ds(w * _LPW, _LPW)])
    pltpu.sync_copy(cg_v, capg_hbm.at[pl.ds(base, MPW)])
    pltpu.sync_copy(f2_v, flat2_hbm.at[pl.ds(base, MPW)])


def _sc_prolog(traffic, cap, pf_flat, pp_flat, ltp_flat):
    k = pl.kernel(
        _sc_prolog_body,
        out_type=(
            jax.ShapeDtypeStruct((L,), jnp.float32),
            jax.ShapeDtypeStruct((M,), jnp.float32),
            jax.ShapeDtypeStruct((M,), jnp.int32),
        ),
        mesh=_mesh(),
        scratch_types=[
            pltpu.VMEM((F,), jnp.float32),
            pltpu.VMEM((L,), jnp.float32),
            pltpu.VMEM((MPW,), jnp.int32),
            pltpu.VMEM((MPW,), jnp.int32),
            pltpu.VMEM((MPW,), jnp.int32),
            pltpu.VMEM((_LPW,), jnp.float32),
            pltpu.VMEM((MPW,), jnp.float32),
            pltpu.VMEM((MPW,), jnp.int32),
        ],
        compiler_params=pltpu.CompilerParams(needs_layout_passes=False),
    )
    return k(traffic, cap, pf_flat, pp_flat, ltp_flat)


def _path_embed_body(tr, pk, psz, ft, p90, bpb, ppb, im, iv, onr,
                     few1, feb1, few2, feb2, path_out):
    w1 = few1[...]
    acc = (tr[...] * w1[0:1, :] + pk[...] * w1[1:2, :] + psz[...] * w1[2:3, :]
           + im[...] * w1[3:4, :] + iv[...] * w1[4:5, :] + onr[...] * w1[5:6, :]
           + ft[:, 0:1] * w1[6:7, :] + ft[:, 1:2] * w1[7:8, :]
           + p90[...] * w1[8:9, :] + ppb[...] * w1[9:10, :]
           + bpb[...] * w1[10:11, :] + feb1[...])
    x = _selu(acc)
    path_out[...] = _selu(
        jnp.dot(x, few2[...], preferred_element_type=jnp.float32) + feb2[...])


def _link_embed_body(cap, ls, lew1, leb1, lew2, leb2, link_out):
    c = cap[...]
    load = ls[...] / (c * 1e9)
    lw1 = lew1[...]
    y = _selu(c * lw1[0:1, :] + load * lw1[1:2, :] + leb1[...])
    link_out[...] = _selu(
        jnp.dot(y, lew2[...], preferred_element_type=jnp.float32) + leb2[...])


def _tc_embed(tr, pk, psz, ft, p90, bpb, ppb, im, iv, onr, cap, ls, fe, le):
    nb = F // BF
    col = pl.BlockSpec((BF, 1), lambda i: (i, 0))
    ft_spec = pl.BlockSpec((BF, 2), lambda i: (i, 0))
    wspecs = [pl.BlockSpec(w.shape, lambda i, n=w.ndim: (0,) * n) for w in fe]
    path = pl.pallas_call(
        _path_embed_body,
        grid=(nb,),
        in_specs=[col] * 3 + [ft_spec] + [col] * 6 + wspecs,
        out_specs=pl.BlockSpec((BF, DIM), lambda i: (i, 0)),
        out_shape=jax.ShapeDtypeStruct((F, DIM), jnp.float32),
    )(tr, pk, psz, ft, p90, bpb, ppb, im, iv, onr, *fe)
    link = pl.pallas_call(
        _link_embed_body,
        out_shape=jax.ShapeDtypeStruct((L, DIM), jnp.float32),
    )(cap, ls, *le)
    return path, link


BF = 2048


def _gru(x, h, wz, wr, wh, hz, hr, hh, bz, br, bh_, cz, cr, ch):
    giz = jnp.dot(x, wz, preferred_element_type=jnp.float32) + bz
    gir = jnp.dot(x, wr, preferred_element_type=jnp.float32) + br
    gih = jnp.dot(x, wh, preferred_element_type=jnp.float32) + bh_
    ghz = jnp.dot(h, hz, preferred_element_type=jnp.float32) + cz
    ghr = jnp.dot(h, hr, preferred_element_type=jnp.float32) + cr
    ghh = jnp.dot(h, hh, preferred_element_type=jnp.float32) + ch
    z = jax.nn.sigmoid(giz + ghz)
    r = jax.nn.sigmoid(gir + ghr)
    c = jnp.tanh(gih + r * ghh)
    return z * h + (1.0 - z) * c


def _scan_body(xs, ps, wz, wr, wh, hz, hr, hh, bz, br, bh_, cz, cr, ch,
               aw, ab, attw_out, ps_out):
    awm = aw[...]
    abv = ab[...]

    def att(q, hq):
        a = jnp.dot(hq, awm, preferred_element_type=jnp.float32) + abv
        a = jnp.where(a >= 0.0, a, a * 0.01)
        e = jnp.exp(a - jnp.max(a, axis=1, keepdims=True))
        attw_out[q] = (e / jnp.sum(e, axis=1, keepdims=True)) * hq

    h = ps[...]
    att(0, h)
    for p_ in range(P):
        h = _gru(xs[p_], h, wz[...], wr[...], wh[...], hz[...], hr[...],
                 hh[...], bz[...], br[...], bh_[...], cz[...], cr[...], ch[...])
        att(p_ + 1, h)
    ps_out[...] = h


def _tc_scan(xs, ps, puw, att):
    nb = F // BF
    wspecs = [pl.BlockSpec((16, 16), lambda i: (0, 0))] * 6 + \
             [pl.BlockSpec((16,), lambda i: (0,))] * 6 + \
             [pl.BlockSpec((16, 16), lambda i: (0, 0)),
              pl.BlockSpec((16,), lambda i: (0,))]
    return pl.pallas_call(
        _scan_body,
        grid=(nb,),
        in_specs=[pl.BlockSpec((P, BF, DIM), lambda i: (0, i, 0)),
                  pl.BlockSpec((BF, DIM), lambda i: (i, 0))] + wspecs,
        out_specs=(pl.BlockSpec((P + 1, BF, DIM), lambda i: (0, i, 0)),
                   pl.BlockSpec((BF, DIM), lambda i: (i, 0))),
        out_shape=(jax.ShapeDtypeStruct((P + 1, F, DIM), jnp.float32),
                   jax.ShapeDtypeStruct((F, DIM), jnp.float32)),
    )(xs, ps, *puw, *att)


def _final_body(xs, ps, wz, wr, wh, hz, hr, hh, bz, br, bh_, cz, cr, ch,
                capg, rw1, rb1, rw2, rb2, rw3, rb3, qd_out):
    h = ps[...]
    qd = jnp.zeros((BF, 1), jnp.float32)
    for p_ in range(P):
        h = _gru(xs[p_], h, wz[...], wr[...], wh[...], hz[...], hr[...],
                 hh[...], bz[...], br[...], bh_[...], cz[...], cr[...], ch[...])
        h1 = _selu(jnp.dot(h, rw1[...], preferred_element_type=jnp.float32)
                         + rb1[...])
        h2 = _selu(jnp.dot(h1, rw2[...], preferred_element_type=jnp.float32)
                         + rb2[...])
        occ = _softplus(jnp.dot(h2, rw3[...],
                                      preferred_element_type=jnp.float32)
                              + rb3[...])
        qd = qd + occ / capg[:, p_:p_ + 1]
    qd_out[...] = qd


def _tc_final(xs, ps, puw, capg, ro):
    nb = F // BF
    wspecs = [pl.BlockSpec((16, 16), lambda i: (0, 0))] * 6 + \
             [pl.BlockSpec((16,), lambda i: (0,))] * 6
    rospecs = [pl.BlockSpec(ro[j].shape, lambda i, n=ro[j].ndim: (0,) * n)
               for j in range(6)]
    return pl.pallas_call(
        _final_body,
        grid=(nb,),
        in_specs=[pl.BlockSpec((P, BF, DIM), lambda i: (0, i, 0)),
                  pl.BlockSpec((BF, DIM), lambda i: (i, 0))] + wspecs +
                 [pl.BlockSpec((BF, P), lambda i: (i, 0))] + rospecs,
        out_specs=pl.BlockSpec((BF, 1), lambda i: (i, 0)),
        out_shape=jax.ShapeDtypeStruct((F, 1), jnp.float32),
    )(xs, ps, *puw, capg, *ro)


BL = 1024


def _lgru_body(rows, ls, wz, wr, wh, hz, hr, hh, bz, br, bh_, cz, cr, ch,
               ls_out):
    r = rows[...]
    s = jnp.zeros((BL, DIM), jnp.float32)
    for d in range(D):
        s = s + r[:, d * DIM:(d + 1) * DIM]
    ls_out[...] = _gru(s, ls[...], wz[...], wr[...], wh[...], hz[...], hr[...],
                       hh[...], bz[...], br[...], bh_[...], cz[...], cr[...],
                       ch[...])


def _tc_linkgru(rows, ls, luw):
    wspecs = [pl.BlockSpec((16, 16), lambda i: (0, 0))] * 6 + \
             [pl.BlockSpec((16,), lambda i: (0,))] * 6
    return pl.pallas_call(
        _lgru_body,
        grid=(L // BL,),
        in_specs=[pl.BlockSpec((BL, D * DIM), lambda i: (i, 0)),
                  pl.BlockSpec((BL, DIM), lambda i: (i, 0))] + wspecs,
        out_specs=pl.BlockSpec((BL, DIM), lambda i: (i, 0)),
        out_shape=jax.ShapeDtypeStruct((L, DIM), jnp.float32),
    )(rows, ls, *luw)


def kernel(flow_traffic, flow_packets, flow_packet_size, flow_type,
           flow_p90PktSize, flow_bitrate_per_burst, flow_pkts_per_burst,
           flow_ipg_mean, flow_ipg_var, flow_on_rate, link_capacity,
           fe_W1, fe_b1, fe_W2, fe_b2,
           le_W1, le_b1, le_W2, le_b2,
           att_W, att_b,
           pu_Wi, pu_Wh, pu_bi, pu_bh,
           lu_Wi, lu_Wh, lu_bi, lu_bh,
           ro_W1, ro_b1, ro_W2, ro_b2, ro_W3, ro_b3,
           link_to_path, path_to_link_flow, path_to_link_pos):
    traffic = flow_traffic.reshape(F)
    cap = link_capacity.reshape(L)
    pf_flat = path_to_link_flow.reshape(M).astype(jnp.int32)
    pp_flat = path_to_link_pos.reshape(M).astype(jnp.int32)
    ltp_flat = link_to_path.reshape(M).astype(jnp.int32)
    ltp_pmaj = jnp.transpose(link_to_path.astype(jnp.int32)).reshape(
        M // CH, CH)

    loadsum, capg, flat2 = _sc_prolog(traffic, cap, pf_flat, pp_flat, ltp_flat)
    flat2 = flat2.reshape(M // CH, CH)
    capg = capg.reshape(F, P)

    def split3(w):
        return tuple(jnp.split(w, 3, axis=-1))

    puw = (*split3(pu_Wi), *split3(pu_Wh), *split3(pu_bi), *split3(pu_bh))
    luw = (*split3(lu_Wi), *split3(lu_Wh), *split3(lu_bi), *split3(lu_bh))
    attp = (att_W, att_b)
    ro = (ro_W1, ro_b1, ro_W2, ro_b2, ro_W3, ro_b3)

    ps, ls = _tc_embed(flow_traffic, flow_packets, flow_packet_size, flow_type,
                       flow_p90PktSize, flow_bitrate_per_burst,
                       flow_pkts_per_burst, flow_ipg_mean, flow_ipg_var,
                       flow_on_rate, link_capacity, loadsum.reshape(L, 1),
                       (fe_W1, fe_b1, fe_W2, fe_b2),
                       (le_W1, le_b1, le_W2, le_b2))

    for it in range(ITERS):
        xs = _sc_gather(ls, ltp_pmaj).reshape(P, F, DIM)
        if it < ITERS - 1:
            attw, ps = _tc_scan(xs, ps, puw, attp)
            rows = _sc_gather(attw.reshape((P + 1) * F, DIM), flat2)
            ls = _tc_linkgru(rows.reshape(L, D * DIM), ls, luw)
        else:
            qd = _tc_final(xs, ps, puw, capg, ro)
    return qd


# speedup vs baseline: 9.2853x; 2.9982x over previous
import functools

import jax
import jax.numpy as jnp
from jax import lax
from jax.experimental import pallas as pl
from jax.experimental.pallas import tpu as pltpu
from jax.experimental.pallas import tpu_sc as plsc

F = 16384
L = 4096
P = 8
D = 32
DIM = 16
ITERS = 12

NC, NS, LANES = 2, 16, 16
NW = NC * NS
M = F * P
MPW = M // NW
CH = 128
NCH = MPW // CH


_SELU_SCALE = 1.0507009873554805
_SELU_ALPHA = 1.6732632423543772


def _selu(x):
    return _SELU_SCALE * jnp.where(x > 0, x, _SELU_ALPHA * (jnp.exp(x) - 1.0))


def _softplus(x):
    return jnp.maximum(x, 0.0) + jnp.log(1.0 + jnp.exp(-jnp.abs(x)))


def _mesh():
    return plsc.VectorSubcoreMesh(core_axis_name="c", subcore_axis_name="s")


def _wid():
    return lax.axis_index("s") * NC + lax.axis_index("c")


def _sc_gather_body(table_hbm, idx_hbm, out_hbm, idx_v, rows_v, sem):
    w = _wid()
    pltpu.sync_copy(idx_hbm.at[pl.ds(w * NCH, NCH)], idx_v)
    cps = [pltpu.async_copy(table_hbm.at[idx_v.at[j]],
                            rows_v.at[pl.ds(j * CH, CH)], sem)
           for j in range(NCH)]
    for c in cps:
        c.wait()
    pltpu.sync_copy(rows_v, out_hbm.at[pl.ds(w * MPW, MPW)])


def _sc_gather(table, idx2d):
    k = pl.kernel(
        _sc_gather_body,
        out_type=jax.ShapeDtypeStruct((M, DIM), jnp.float32),
        mesh=_mesh(),
        scratch_types=[
            pltpu.VMEM((NCH, CH), jnp.int32),
            pltpu.VMEM((MPW, DIM), jnp.float32),
            pltpu.SemaphoreType.DMA,
        ],
        compiler_params=pltpu.CompilerParams(use_tc_tiling_on_sc=False),
    )
    return k(table, idx2d)


_LPW = L // NW


def _sc_prolog_body(tr_hbm, cap_hbm, pf_hbm, pp_hbm, ltp_hbm,
                    loadsum_hbm, capg_hbm, flat2_hbm,
                    tr_v, cap_v, pf_v, pp_v, ltp_v, ls_v, cg_v, f2_v):
    w = _wid()
    base = w * MPW
    pltpu.sync_copy(tr_hbm, tr_v)
    pltpu.sync_copy(cap_hbm, cap_v)
    pltpu.sync_copy(pf_hbm.at[pl.ds(base, MPW)], pf_v)
    pltpu.sync_copy(pp_hbm.at[pl.ds(base, MPW)], pp_v)
    pltpu.sync_copy(ltp_hbm.at[pl.ds(base, MPW)], ltp_v)

    lanes = lax.iota(jnp.int32, LANES)
    for g in range(_LPW // LANES):
        acc = jnp.zeros((LANES,), jnp.float32)
        for d in range(D):
            pos = lanes * D + (g * LANES * D + d)
            fidx = plsc.load_gather(pf_v, [pos])
            acc = acc + plsc.load_gather(tr_v, [fidx])
        ls_v[pl.ds(g * LANES, LANES)] = acc
    for k in range(MPW // LANES):
        sl = pl.ds(k * LANES, LANES)
        li = ltp_v[sl]
        cg_v[sl] = plsc.load_gather(cap_v, [li])
        f2_v[sl] = pp_v[sl] * F + pf_v[sl]

    pltpu.sync_copy(ls_v, loadsum_hbm.at[pl.ds(w * _LPW, _LPW)])
    pltpu.sync_copy(cg_v, capg_hbm.at[pl.ds(base, MPW)])
    pltpu.sync_copy(f2_v, flat2_hbm.at[pl.ds(base, MPW)])


def _sc_prolog(traffic, cap, pf_flat, pp_flat, ltp_flat):
    k = pl.kernel(
        _sc_prolog_body,
        out_type=(
            jax.ShapeDtypeStruct((L,), jnp.float32),
            jax.ShapeDtypeStruct((M,), jnp.float32),
            jax.ShapeDtypeStruct((M,), jnp.int32),
        ),
        mesh=_mesh(),
        scratch_types=[
            pltpu.VMEM((F,), jnp.float32),
            pltpu.VMEM((L,), jnp.float32),
            pltpu.VMEM((MPW,), jnp.int32),
            pltpu.VMEM((MPW,), jnp.int32),
            pltpu.VMEM((MPW,), jnp.int32),
            pltpu.VMEM((_LPW,), jnp.float32),
            pltpu.VMEM((MPW,), jnp.float32),
            pltpu.VMEM((MPW,), jnp.int32),
        ],
        compiler_params=pltpu.CompilerParams(needs_layout_passes=False),
    )
    return k(traffic, cap, pf_flat, pp_flat, ltp_flat)


def _path_embed_body(tr, pk, psz, ft, p90, bpb, ppb, im, iv, onr,
                     few1, feb1, few2, feb2, path_out):
    w1 = few1[...]
    acc = (tr[...] * w1[0:1, :] + pk[...] * w1[1:2, :] + psz[...] * w1[2:3, :]
           + im[...] * w1[3:4, :] + iv[...] * w1[4:5, :] + onr[...] * w1[5:6, :]
           + ft[:, 0:1] * w1[6:7, :] + ft[:, 1:2] * w1[7:8, :]
           + p90[...] * w1[8:9, :] + ppb[...] * w1[9:10, :]
           + bpb[...] * w1[10:11, :] + feb1[...])
    x = _selu(acc)
    path_out[...] = _selu(
        jnp.dot(x, few2[...], preferred_element_type=jnp.float32) + feb2[...])


def _link_embed_body(cap, ls, lew1, leb1, lew2, leb2, link_out):
    c = cap[...]
    load = ls[...] / (c * 1e9)
    lw1 = lew1[...]
    y = _selu(c * lw1[0:1, :] + load * lw1[1:2, :] + leb1[...])
    link_out[...] = _selu(
        jnp.dot(y, lew2[...], preferred_element_type=jnp.float32) + leb2[...])


def _tc_embed(tr, pk, psz, ft, p90, bpb, ppb, im, iv, onr, cap, ls, fe, le):
    nb = F // BF
    col = pl.BlockSpec((BF, 1), lambda i: (i, 0))
    ft_spec = pl.BlockSpec((BF, 2), lambda i: (i, 0))
    wspecs = [pl.BlockSpec(w.shape, lambda i, n=w.ndim: (0,) * n) for w in fe]
    path = pl.pallas_call(
        _path_embed_body,
        grid=(nb,),
        in_specs=[col] * 3 + [ft_spec] + [col] * 6 + wspecs,
        out_specs=pl.BlockSpec((BF, DIM), lambda i: (i, 0)),
        out_shape=jax.ShapeDtypeStruct((F, DIM), jnp.float32),
    )(tr, pk, psz, ft, p90, bpb, ppb, im, iv, onr, *fe)
    link = pl.pallas_call(
        _link_embed_body,
        out_shape=jax.ShapeDtypeStruct((L, DIM), jnp.float32),
    )(cap, ls, *le)
    return path, link


BF = 2048
F8 = F // 8
L8 = L // 8
W = 128


def _gru(x, h, wz, wr, wh, hz, hr, hh, bz, br, bh_, cz, cr, ch):
    giz = jnp.dot(x, wz, preferred_element_type=jnp.float32) + bz
    gir = jnp.dot(x, wr, preferred_element_type=jnp.float32) + br
    gih = jnp.dot(x, wh, preferred_element_type=jnp.float32) + bh_
    ghz = jnp.dot(h, hz, preferred_element_type=jnp.float32) + cz
    ghr = jnp.dot(h, hr, preferred_element_type=jnp.float32) + cr
    ghh = jnp.dot(h, hh, preferred_element_type=jnp.float32) + ch
    z = jax.nn.sigmoid(giz + ghz)
    r = jax.nn.sigmoid(gir + ghr)
    c = jnp.tanh(gih + r * ghh)
    return z * h + (1.0 - z) * c


def _gru_wide(x, h, w_all, b_all):
    xh = jnp.concatenate([x, h], axis=1)
    g = jnp.dot(xh, w_all, preferred_element_type=jnp.float32) + b_all
    z = jax.nn.sigmoid(g[:, 0:W])
    r = jax.nn.sigmoid(g[:, W:2 * W])
    c = jnp.tanh(g[:, 2 * W:3 * W] + r * g[:, 3 * W:4 * W])
    return z * h + (1.0 - z) * c


def _scan_body(xs, ps, w_all, b_all, aw, ab, gsum, attw_out, ps_out):
    awm = aw[...]
    abv = ab[...]
    gs = gsum[...]
    wa = w_all[...]
    ba = b_all[...]

    def att(q, hq):
        a = jnp.dot(hq, awm, preferred_element_type=jnp.float32) + abv
        a = jnp.where(a >= 0.0, a, a * 0.01)
        e = jnp.exp(a - jnp.max(a, axis=1, keepdims=True))
        s = jnp.dot(e, gs, preferred_element_type=jnp.float32)
        attw_out[q] = (e / jnp.maximum(s, 1e-30)) * hq

    h = ps[...]
    att(0, h)
    for p_ in range(P):
        h = _gru_wide(xs[p_], h, wa, ba)
        att(p_ + 1, h)
    ps_out[...] = h


def _tc_scan(xs, ps, w_all, b_all, attp):
    return pl.pallas_call(
        _scan_body,
        out_shape=(jax.ShapeDtypeStruct((P + 1, F8, W), jnp.float32),
                   jax.ShapeDtypeStruct((F8, W), jnp.float32)),
    )(xs, ps, w_all, b_all, *attp)


def _final_body(xs, ps, w_all, b_all, capg, rw1, rb1, rw2, rb2, rw3, rb3,
                qd_out):
    wa = w_all[...]
    ba = b_all[...]
    h = ps[...]
    qd = jnp.zeros((F8, 8), jnp.float32)
    for p_ in range(P):
        h = _gru_wide(xs[p_], h, wa, ba)
        h1 = _selu(jnp.dot(h, rw1[...], preferred_element_type=jnp.float32)
                   + rb1[...])
        h2 = _selu(jnp.dot(h1, rw2[...], preferred_element_type=jnp.float32)
                   + rb2[...])
        occ = _softplus(jnp.dot(h2, rw3[...],
                                preferred_element_type=jnp.float32)
                        + rb3[...])
        qd = qd + occ / capg[p_]
    qd_out[...] = qd


def _tc_final(xs, ps, w_all, b_all, capg, row):
    return pl.pallas_call(
        _final_body,
        out_shape=jax.ShapeDtypeStruct((F8, 8), jnp.float32),
    )(xs, ps, w_all, b_all, capg, *row)


BL = 1024


def _lgru_body(rows, ls, wz, wr, wh, hz, hr, hh, bz, br, bh_, cz, cr, ch,
               ls_out):
    r = rows[...]
    s = jnp.zeros((BL, DIM), jnp.float32)
    for d in range(D):
        s = s + r[:, d * DIM:(d + 1) * DIM]
    ls_out[...] = _gru(s, ls[...], wz[...], wr[...], wh[...], hz[...], hr[...],
                       hh[...], bz[...], br[...], bh_[...], cz[...], cr[...],
                       ch[...])


def _tc_linkgru(rows, ls, luw):
    wspecs = [pl.BlockSpec((16, 16), lambda i: (0, 0))] * 6 + \
             [pl.BlockSpec((16,), lambda i: (0,))] * 6
    return pl.pallas_call(
        _lgru_body,
        grid=(L // BL,),
        in_specs=[pl.BlockSpec((BL, D * DIM), lambda i: (i, 0)),
                  pl.BlockSpec((BL, DIM), lambda i: (i, 0))] + wspecs,
        out_specs=pl.BlockSpec((BL, DIM), lambda i: (i, 0)),
        out_shape=jax.ShapeDtypeStruct((L, DIM), jnp.float32),
    )(rows, ls, *luw)


def kernel(flow_traffic, flow_packets, flow_packet_size, flow_type,
           flow_p90PktSize, flow_bitrate_per_burst, flow_pkts_per_burst,
           flow_ipg_mean, flow_ipg_var, flow_on_rate, link_capacity,
           fe_W1, fe_b1, fe_W2, fe_b2,
           le_W1, le_b1, le_W2, le_b2,
           att_W, att_b,
           pu_Wi, pu_Wh, pu_bi, pu_bh,
           lu_Wi, lu_Wh, lu_bi, lu_bh,
           ro_W1, ro_b1, ro_W2, ro_b2, ro_W3, ro_b3,
           link_to_path, path_to_link_flow, path_to_link_pos):
    traffic = flow_traffic.reshape(F)
    cap = link_capacity.reshape(L)
    pf_flat = path_to_link_flow.reshape(M).astype(jnp.int32)
    pp_flat = path_to_link_pos.reshape(M).astype(jnp.int32)
    ltp_flat = link_to_path.reshape(M).astype(jnp.int32)
    ltp_pmaj = jnp.transpose(link_to_path.astype(jnp.int32)).reshape(
        M // CH, CH)

    loadsum, capg, flat2 = _sc_prolog(traffic, cap, pf_flat, pp_flat, ltp_flat)
    flat2 = flat2.reshape(M // CH, CH)
    capg_w = jnp.transpose(capg.reshape(F, P)).reshape(P, F8, 8)

    eye8 = jnp.eye(8, dtype=jnp.float32)

    def bd(w):
        return jnp.kron(eye8, w)

    def t8(b):
        return jnp.tile(b, 8)

    def split3(w):
        return tuple(jnp.split(w, 3, axis=-1))

    wiz, wir, wih = split3(pu_Wi)
    whz, whr, whh = split3(pu_Wh)
    biz, bir, bih = split3(pu_bi)
    bhz, bhr, bhh = split3(pu_bh)
    zcol = jnp.zeros((W, W), jnp.float32)
    w_all = jnp.concatenate([
        jnp.concatenate([bd(wiz), bd(wir), bd(wih), zcol], axis=1),
        jnp.concatenate([bd(whz), bd(whr), zcol, bd(whh)], axis=1),
    ], axis=0)
    b_all = jnp.concatenate([t8(biz + bhz), t8(bir + bhr), t8(bih), t8(bhh)])

    attp = (bd(att_W), t8(att_b),
            jnp.kron(eye8, jnp.ones((DIM, DIM), jnp.float32)))
    row = (bd(ro_W1), t8(ro_b1), bd(ro_W2), t8(ro_b2), bd(ro_W3), t8(ro_b3))
    luw = (*split3(lu_Wi), *split3(lu_Wh), *split3(lu_bi), *split3(lu_bh))

    ps, ls = _tc_embed(flow_traffic, flow_packets, flow_packet_size, flow_type,
                       flow_p90PktSize, flow_bitrate_per_burst,
                       flow_pkts_per_burst, flow_ipg_mean, flow_ipg_var,
                       flow_on_rate, link_capacity, loadsum.reshape(L, 1),
                       (fe_W1, fe_b1, fe_W2, fe_b2),
                       (le_W1, le_b1, le_W2, le_b2))
    ps_w = ps.reshape(F8, W)

    for it in range(ITERS):
        xs = _sc_gather(ls, ltp_pmaj).reshape(P, F8, W)
        if it < ITERS - 1:
            attw, ps_w = _tc_scan(xs, ps_w, w_all, b_all, attp)
            rows = _sc_gather(attw.reshape((P + 1) * F, DIM), flat2)
            ls = _tc_linkgru(rows.reshape(L, D * DIM), ls, luw)
        else:
            qd = _tc_final(xs, ps_w, w_all, b_all, capg_w, row)
    return qd.reshape(F, 1)


# speedup vs baseline: 11.7109x; 1.2612x over previous
import functools

import jax
import jax.numpy as jnp
from jax import lax
from jax.experimental import pallas as pl
from jax.experimental.pallas import tpu as pltpu
from jax.experimental.pallas import tpu_sc as plsc

F = 16384
L = 4096
P = 8
D = 32
DIM = 16
ITERS = 12

NC, NS, LANES = 2, 16, 16
NW = NC * NS
M = F * P
MPW = M // NW
CH = 128
NCH = MPW // CH


_SELU_SCALE = 1.0507009873554805
_SELU_ALPHA = 1.6732632423543772


def _selu(x):
    return _SELU_SCALE * jnp.where(x > 0, x, _SELU_ALPHA * (jnp.exp(x) - 1.0))


def _softplus(x):
    return jnp.maximum(x, 0.0) + jnp.log(1.0 + jnp.exp(-jnp.abs(x)))


def _mesh():
    return plsc.VectorSubcoreMesh(core_axis_name="c", subcore_axis_name="s")


def _wid():
    return lax.axis_index("s") * NC + lax.axis_index("c")


def _sc_gather_body(table_hbm, idx_hbm, out_hbm, idx_v, rows_v, sem):
    w = _wid()
    pltpu.sync_copy(idx_hbm.at[pl.ds(w * NCH, NCH)], idx_v)
    cps = [pltpu.async_copy(table_hbm.at[idx_v.at[j]],
                            rows_v.at[pl.ds(j * CH, CH)], sem)
           for j in range(NCH)]
    for c in cps:
        c.wait()
    pltpu.sync_copy(rows_v, out_hbm.at[pl.ds(w * MPW, MPW)])


def _sc_gather(table, idx2d):
    k = pl.kernel(
        _sc_gather_body,
        out_type=jax.ShapeDtypeStruct((M, DIM), jnp.float32),
        mesh=_mesh(),
        scratch_types=[
            pltpu.VMEM((NCH, CH), jnp.int32),
            pltpu.VMEM((MPW, DIM), jnp.float32),
            pltpu.SemaphoreType.DMA,
        ],
        compiler_params=pltpu.CompilerParams(use_tc_tiling_on_sc=False),
    )
    return k(table, idx2d)


def _sc_gather_sum_body(table_hbm, idx_hbm, out_hbm, idx_v, rows_v, score_v,
                        sem):
    w = _wid()
    pltpu.sync_copy(idx_hbm.at[pl.ds(w * NCH, NCH)], idx_v)
    cps = [pltpu.async_copy(table_hbm.at[idx_v.at[j]],
                            rows_v.at[pl.ds(j * CH, CH)], sem)
           for j in range(NCH)]
    for c in cps:
        c.wait()

    def red(l, _):
        acc = rows_v[l * D]
        for d_ in range(1, D):
            acc = acc + rows_v[l * D + d_]
        score_v[l] = acc
        return _

    lax.fori_loop(0, _LPW, red, None)
    pltpu.sync_copy(score_v, out_hbm.at[pl.ds(w * _LPW, _LPW)])


def _sc_gather_sum(table, idx2d):
    k = pl.kernel(
        _sc_gather_sum_body,
        out_type=jax.ShapeDtypeStruct((L, DIM), jnp.float32),
        mesh=_mesh(),
        scratch_types=[
            pltpu.VMEM((NCH, CH), jnp.int32),
            pltpu.VMEM((MPW, DIM), jnp.float32),
            pltpu.VMEM((_LPW, DIM), jnp.float32),
            pltpu.SemaphoreType.DMA,
        ],
        compiler_params=pltpu.CompilerParams(use_tc_tiling_on_sc=False,
                                             needs_layout_passes=False),
    )
    return k(table, idx2d)


_LPW = L // NW


def _sc_prolog_body(tr_hbm, cap_hbm, pf_hbm, pp_hbm, ltp_hbm,
                    loadsum_hbm, capg_hbm, flat2_hbm,
                    tr_v, cap_v, pf_v, pp_v, ltp_v, ls_v, cg_v, f2_v):
    w = _wid()
    base = w * MPW
    pltpu.sync_copy(tr_hbm, tr_v)
    pltpu.sync_copy(cap_hbm, cap_v)
    pltpu.sync_copy(pf_hbm.at[pl.ds(base, MPW)], pf_v)
    pltpu.sync_copy(pp_hbm.at[pl.ds(base, MPW)], pp_v)
    pltpu.sync_copy(ltp_hbm.at[pl.ds(base, MPW)], ltp_v)

    lanes = lax.iota(jnp.int32, LANES)
    for g in range(_LPW // LANES):
        acc = jnp.zeros((LANES,), jnp.float32)
        for d in range(D):
            pos = lanes * D + (g * LANES * D + d)
            fidx = plsc.load_gather(pf_v, [pos])
            acc = acc + plsc.load_gather(tr_v, [fidx])
        ls_v[pl.ds(g * LANES, LANES)] = acc
    for k in range(MPW // LANES):
        sl = pl.ds(k * LANES, LANES)
        li = ltp_v[sl]
        cg_v[sl] = plsc.load_gather(cap_v, [li])
        f2_v[sl] = pp_v[sl] * F + pf_v[sl]

    pltpu.sync_copy(ls_v, loadsum_hbm.at[pl.ds(w * _LPW, _LPW)])
    pltpu.sync_copy(cg_v, capg_hbm.at[pl.ds(base, MPW)])
    pltpu.sync_copy(f2_v, flat2_hbm.at[pl.ds(base, MPW)])


def _sc_prolog(traffic, cap, pf_flat, pp_flat, ltp_flat):
    k = pl.kernel(
        _sc_prolog_body,
        out_type=(
            jax.ShapeDtypeStruct((L,), jnp.float32),
            jax.ShapeDtypeStruct((M,), jnp.float32),
            jax.ShapeDtypeStruct((M,), jnp.int32),
        ),
        mesh=_mesh(),
        scratch_types=[
            pltpu.VMEM((F,), jnp.float32),
            pltpu.VMEM((L,), jnp.float32),
            pltpu.VMEM((MPW,), jnp.int32),
            pltpu.VMEM((MPW,), jnp.int32),
            pltpu.VMEM((MPW,), jnp.int32),
            pltpu.VMEM((_LPW,), jnp.float32),
            pltpu.VMEM((MPW,), jnp.float32),
            pltpu.VMEM((MPW,), jnp.int32),
        ],
        compiler_params=pltpu.CompilerParams(needs_layout_passes=False),
    )
    return k(traffic, cap, pf_flat, pp_flat, ltp_flat)


def _path_embed_body(tr, pk, psz, ft, p90, bpb, ppb, im, iv, onr,
                     few1, feb1, few2, feb2, path_out):
    w1 = few1[...]
    acc = (tr[...] * w1[0:1, :] + pk[...] * w1[1:2, :] + psz[...] * w1[2:3, :]
           + im[...] * w1[3:4, :] + iv[...] * w1[4:5, :] + onr[...] * w1[5:6, :]
           + ft[:, 0:1] * w1[6:7, :] + ft[:, 1:2] * w1[7:8, :]
           + p90[...] * w1[8:9, :] + ppb[...] * w1[9:10, :]
           + bpb[...] * w1[10:11, :] + feb1[...])
    x = _selu(acc)
    path_out[...] = _selu(
        jnp.dot(x, few2[...], preferred_element_type=jnp.float32) + feb2[...])


def _link_embed_body(cap, ls, lew1, leb1, lew2, leb2, link_out):
    c = cap[...]
    load = ls[...] / (c * 1e9)
    lw1 = lew1[...]
    y = _selu(c * lw1[0:1, :] + load * lw1[1:2, :] + leb1[...])
    link_out[...] = _selu(
        jnp.dot(y, lew2[...], preferred_element_type=jnp.float32) + leb2[...])


def _tc_embed(tr, pk, psz, ft, p90, bpb, ppb, im, iv, onr, cap, ls, fe, le):
    nb = F // BF
    col = pl.BlockSpec((BF, 1), lambda i: (i, 0))
    ft_spec = pl.BlockSpec((BF, 2), lambda i: (i, 0))
    wspecs = [pl.BlockSpec(w.shape, lambda i, n=w.ndim: (0,) * n) for w in fe]
    path = pl.pallas_call(
        _path_embed_body,
        grid=(nb,),
        in_specs=[col] * 3 + [ft_spec] + [col] * 6 + wspecs,
        out_specs=pl.BlockSpec((BF, DIM), lambda i: (i, 0)),
        out_shape=jax.ShapeDtypeStruct((F, DIM), jnp.float32),
    )(tr, pk, psz, ft, p90, bpb, ppb, im, iv, onr, *fe)
    link = pl.pallas_call(
        _link_embed_body,
        out_shape=jax.ShapeDtypeStruct((L, DIM), jnp.float32),
    )(cap, ls, *le)
    return path, link


BF = 2048
F8 = F // 8
L8 = L // 8
W = 128


def _gru(x, h, wz, wr, wh, hz, hr, hh, bz, br, bh_, cz, cr, ch):
    giz = jnp.dot(x, wz, preferred_element_type=jnp.float32) + bz
    gir = jnp.dot(x, wr, preferred_element_type=jnp.float32) + br
    gih = jnp.dot(x, wh, preferred_element_type=jnp.float32) + bh_
    ghz = jnp.dot(h, hz, preferred_element_type=jnp.float32) + cz
    ghr = jnp.dot(h, hr, preferred_element_type=jnp.float32) + cr
    ghh = jnp.dot(h, hh, preferred_element_type=jnp.float32) + ch
    z = jax.nn.sigmoid(giz + ghz)
    r = jax.nn.sigmoid(gir + ghr)
    c = jnp.tanh(gih + r * ghh)
    return z * h + (1.0 - z) * c


def _gru_wide(x, h, w_all, b_all):
    xh = jnp.concatenate([x, h], axis=1)
    g = jnp.dot(xh, w_all, preferred_element_type=jnp.float32) + b_all
    z = jax.nn.sigmoid(g[:, 0:W])
    r = jax.nn.sigmoid(g[:, W:2 * W])
    c = jnp.tanh(g[:, 2 * W:3 * W] + r * g[:, 3 * W:4 * W])
    return z * h + (1.0 - z) * c


def _scan_body(xs, ps, w_all, b_all, aw, ab, gsum, attw_out, ps_out):
    awm = aw[...]
    abv = ab[...]
    gs = gsum[...]
    wa = w_all[...]
    ba = b_all[...]

    def att(q, hq):
        a = jnp.dot(hq, awm, preferred_element_type=jnp.float32) + abv
        a = jnp.where(a >= 0.0, a, a * 0.01)
        e = jnp.exp(a - jnp.max(a, axis=1, keepdims=True))
        s = jnp.dot(e, gs, preferred_element_type=jnp.float32)
        attw_out[q] = (e / jnp.maximum(s, 1e-30)) * hq

    h = ps[...]
    att(0, h)
    for p_ in range(P):
        h = _gru_wide(xs[p_], h, wa, ba)
        att(p_ + 1, h)
    ps_out[...] = h


def _tc_scan(xs, ps, w_all, b_all, attp):
    return pl.pallas_call(
        _scan_body,
        out_shape=(jax.ShapeDtypeStruct((P + 1, F8, W), jnp.float32),
                   jax.ShapeDtypeStruct((F8, W), jnp.float32)),
    )(xs, ps, w_all, b_all, *attp)


def _final_body(xs, ps, w_all, b_all, capg, rw1, rb1, rw2, rb2, rw3, rb3,
                qd_out):
    wa = w_all[...]
    ba = b_all[...]
    h = ps[...]
    qd = jnp.zeros((F8, 8), jnp.float32)
    for p_ in range(P):
        h = _gru_wide(xs[p_], h, wa, ba)
        h1 = _selu(jnp.dot(h, rw1[...], preferred_element_type=jnp.float32)
                   + rb1[...])
        h2 = _selu(jnp.dot(h1, rw2[...], preferred_element_type=jnp.float32)
                   + rb2[...])
        occ = _softplus(jnp.dot(h2, rw3[...],
                                preferred_element_type=jnp.float32)
                        + rb3[...])
        qd = qd + occ / capg[p_]
    qd_out[...] = qd


def _tc_final(xs, ps, w_all, b_all, capg, row):
    return pl.pallas_call(
        _final_body,
        out_shape=jax.ShapeDtypeStruct((F8, 8), jnp.float32),
    )(xs, ps, w_all, b_all, capg, *row)


def _lgru_body(score, ls, wz, wr, wh, hz, hr, hh, bz, br, bh_, cz, cr, ch,
               ls_out):
    ls_out[...] = _gru(score[...], ls[...], wz[...], wr[...], wh[...],
                       hz[...], hr[...], hh[...], bz[...], br[...], bh_[...],
                       cz[...], cr[...], ch[...])


def _tc_linkgru(score, ls, luw):
    return pl.pallas_call(
        _lgru_body,
        out_shape=jax.ShapeDtypeStruct((L, DIM), jnp.float32),
    )(score, ls, *luw)


def kernel(flow_traffic, flow_packets, flow_packet_size, flow_type,
           flow_p90PktSize, flow_bitrate_per_burst, flow_pkts_per_burst,
           flow_ipg_mean, flow_ipg_var, flow_on_rate, link_capacity,
           fe_W1, fe_b1, fe_W2, fe_b2,
           le_W1, le_b1, le_W2, le_b2,
           att_W, att_b,
           pu_Wi, pu_Wh, pu_bi, pu_bh,
           lu_Wi, lu_Wh, lu_bi, lu_bh,
           ro_W1, ro_b1, ro_W2, ro_b2, ro_W3, ro_b3,
           link_to_path, path_to_link_flow, path_to_link_pos):
    traffic = flow_traffic.reshape(F)
    cap = link_capacity.reshape(L)
    pf_flat = path_to_link_flow.reshape(M).astype(jnp.int32)
    pp_flat = path_to_link_pos.reshape(M).astype(jnp.int32)
    ltp_flat = link_to_path.reshape(M).astype(jnp.int32)
    ltp_pmaj = jnp.transpose(link_to_path.astype(jnp.int32)).reshape(
        M // CH, CH)

    loadsum, capg, flat2 = _sc_prolog(traffic, cap, pf_flat, pp_flat, ltp_flat)
    flat2 = flat2.reshape(M // CH, CH)
    capg_w = jnp.transpose(capg.reshape(F, P)).reshape(P, F8, 8)

    eye8 = jnp.eye(8, dtype=jnp.float32)

    def bd(w):
        return jnp.kron(eye8, w)

    def t8(b):
        return jnp.tile(b, 8)

    def split3(w):
        return tuple(jnp.split(w, 3, axis=-1))

    wiz, wir, wih = split3(pu_Wi)
    whz, whr, whh = split3(pu_Wh)
    biz, bir, bih = split3(pu_bi)
    bhz, bhr, bhh = split3(pu_bh)
    zcol = jnp.zeros((W, W), jnp.float32)
    w_all = jnp.concatenate([
        jnp.concatenate([bd(wiz), bd(wir), bd(wih), zcol], axis=1),
        jnp.concatenate([bd(whz), bd(whr), zcol, bd(whh)], axis=1),
    ], axis=0)
    b_all = jnp.concatenate([t8(biz + bhz), t8(bir + bhr), t8(bih), t8(bhh)])

    attp = (bd(att_W), t8(att_b),
            jnp.kron(eye8, jnp.ones((DIM, DIM), jnp.float32)))
    row = (bd(ro_W1), t8(ro_b1), bd(ro_W2), t8(ro_b2), bd(ro_W3), t8(ro_b3))
    luw = (*split3(lu_Wi), *split3(lu_Wh), *split3(lu_bi), *split3(lu_bh))

    ps, ls = _tc_embed(flow_traffic, flow_packets, flow_packet_size, flow_type,
                       flow_p90PktSize, flow_bitrate_per_burst,
                       flow_pkts_per_burst, flow_ipg_mean, flow_ipg_var,
                       flow_on_rate, link_capacity, loadsum.reshape(L, 1),
                       (fe_W1, fe_b1, fe_W2, fe_b2),
                       (le_W1, le_b1, le_W2, le_b2))
    ps_w = ps.reshape(F8, W)

    for it in range(ITERS):
        xs = _sc_gather(ls, ltp_pmaj).reshape(P, F8, W)
        if it < ITERS - 1:
            attw, ps_w = _tc_scan(xs, ps_w, w_all, b_all, attp)
            score = _sc_gather_sum(attw.reshape((P + 1) * F, DIM), flat2)
            ls = _tc_linkgru(score, ls, luw)
        else:
            qd = _tc_final(xs, ps_w, w_all, b_all, capg_w, row)
    return qd.reshape(F, 1)
